# Initial kernel scaffold; baseline (speedup 1.0000x reference)
import functools

import jax
import jax.numpy as jnp
from jax import lax
from jax.experimental import pallas as pl
from jax.experimental.pallas import tpu as pltpu
from jax.experimental.pallas import tpu_sc as plsc

_N = 10000
_D = 128
_NS = 16
_NC = 2
_K = 128
_ROWS_PER_TILE = _N // _NS
_ROW_CHUNK = 125


def _cdiv(a, b):
    return (a + b - 1) // b


_RB = 1000
_GRID = _N // _RB


def _tc0_body(f_ref, wl_ref, bl_ref, w1_ref, p_ref, h_ref):
    p = jnp.dot(f_ref[...], wl_ref[...], preferred_element_type=jnp.float32)
    p = p + bl_ref[...]
    p_ref[...] = p
    x = jax.nn.sigmoid(p)
    h = jnp.dot(x, w1_ref[...], preferred_element_type=jnp.float32)
    h_ref[0] = h[:, :_D]
    h_ref[1] = h[:, _D:]


def _tc0(features, W_lin, b_lin, W1):
    return pl.pallas_call(
        _tc0_body,
        grid=(_GRID,),
        in_specs=[
            pl.BlockSpec((_RB, 128), lambda i: (i, 0)),
            pl.BlockSpec((128, 256), lambda i: (0, 0)),
            pl.BlockSpec((1, 256), lambda i: (0, 0)),
            pl.BlockSpec((256, 256), lambda i: (0, 0)),
        ],
        out_specs=[
            pl.BlockSpec((_RB, 256), lambda i: (i, 0)),
            pl.BlockSpec((2, _RB, _D), lambda i: (0, i, 0)),
        ],
        out_shape=[
            jax.ShapeDtypeStruct((_N, 256), jnp.float32),
            jax.ShapeDtypeStruct((2, _N, _D), jnp.float32),
        ],
    )(features, W_lin, b_lin.reshape(1, 256), W1)


def _tc_mid_body(s_ref, b_ref, w_ref, g_ref, h_ref):
    g = jnp.concatenate([s_ref[0], s_ref[1]], axis=1) + b_ref[...]
    g_ref[...] = g
    x = jnp.maximum(g, 0.0)
    h = jnp.dot(x, w_ref[...], preferred_element_type=jnp.float32)
    h_ref[0] = h[:, :_D]
    h_ref[1] = h[:, _D:]


def _tc_mid(s_split, b, W):
    return pl.pallas_call(
        _tc_mid_body,
        grid=(_GRID,),
        in_specs=[
            pl.BlockSpec((2, _RB, _D), lambda i: (0, i, 0)),
            pl.BlockSpec((1, 256), lambda i: (0, 0)),
            pl.BlockSpec((256, 256), lambda i: (0, 0)),
        ],
        out_specs=[
            pl.BlockSpec((_RB, 256), lambda i: (i, 0)),
            pl.BlockSpec((2, _RB, _D), lambda i: (0, i, 0)),
        ],
        out_shape=[
            jax.ShapeDtypeStruct((_N, 256), jnp.float32),
            jax.ShapeDtypeStruct((2, _N, _D), jnp.float32),
        ],
    )(s_split, b.reshape(1, 256), W)


def _tc_bias_body(s_ref, b_ref, g_ref):
    g_ref[...] = jnp.concatenate([s_ref[0], s_ref[1]], axis=1) + b_ref[...]


def _tc_bias(s_split, b):
    return pl.pallas_call(
        _tc_bias_body,
        grid=(_GRID,),
        in_specs=[
            pl.BlockSpec((2, _RB, _D), lambda i: (0, i, 0)),
            pl.BlockSpec((1, 256), lambda i: (0, 0)),
        ],
        out_specs=pl.BlockSpec((_RB, 256), lambda i: (i, 0)),
        out_shape=jax.ShapeDtypeStruct((_N, 256), jnp.float32),
    )(s_split, b.reshape(1, 256))


def _spmm_body(h_hbm, src_hbm, dst_hbm, w_hbm, out_hbm,
               acc, src_v, dst_v, w_v, rows_v, sem, nchunk):
    c = lax.axis_index("c")
    s = lax.axis_index("s")

    zero = jnp.zeros((16,), jnp.float32)

    def zrow(r, carry):
        for v in range(8):
            rows_v[r, pl.ds(v * 16, 16)] = zero
        return carry

    lax.fori_loop(0, _K, zrow, 0)
    for kk in range(_ROWS_PER_TILE // _ROW_CHUNK):
        pltpu.sync_copy(
            rows_v.at[pl.ds(0, _ROW_CHUNK)],
            acc.at[pl.ds(s * _ROWS_PER_TILE + kk * _ROW_CHUNK, _ROW_CHUNK)],
        )
    plsc.subcore_barrier()

    pltpu.sync_copy(src_hbm.at[c, s], src_v)
    pltpu.sync_copy(dst_hbm.at[s], dst_v)
    pltpu.sync_copy(w_hbm.at[s], w_v)

    def chunk(j, carry):
        pltpu.async_copy(h_hbm.at[src_v.at[j]], rows_v, sem).wait()

        def edge(e, c2):
            ws = w_v[j, e]
            for v in range(8):
                sl = pl.ds(v * 16, 16)
                rows_v[e, sl] = rows_v[e, sl] * ws
            return c2

        lax.fori_loop(0, _K, edge, 0)
        pltpu.sync_copy(rows_v, acc.at[dst_v.at[j]], add=True)
        return carry

    lax.fori_loop(0, nchunk, chunk, 0)
    plsc.subcore_barrier()

    pltpu.sync_copy(
        acc.at[pl.ds(s * _ROWS_PER_TILE, _ROWS_PER_TILE)],
        out_hbm.at[c, pl.ds(s * _ROWS_PER_TILE, _ROWS_PER_TILE)],
    )


def _make_spmm(nchunk):
    return functools.partial(
        pl.kernel,
        out_type=jax.ShapeDtypeStruct((_NC, _N, _D), jnp.float32),
        mesh=plsc.VectorSubcoreMesh(core_axis_name="c", subcore_axis_name="s"),
        scratch_types=[
            pltpu.VMEM_SHARED((_N, _D), jnp.float32),
            pltpu.VMEM((nchunk, _K), jnp.int32),
            pltpu.VMEM((nchunk, _K), jnp.int32),
            pltpu.VMEM((nchunk, _K), jnp.float32),
            pltpu.VMEM((_K, _D), jnp.float32),
            pltpu.SemaphoreType.DMA,
        ],
    )(functools.partial(_spmm_body, nchunk=nchunk))


def kernel(features, edge_index, edge_weight, W_lin, b_lin, W1, b1, W2, b2):
    n = features.shape[0]
    e = edge_index.shape[1]
    nchunk = _cdiv(e, _NS * _K)
    e_pad = _NS * nchunk * _K

    src = edge_index[0]
    dst = edge_index[1]
    pad = e_pad - e
    if pad:
        zi = jnp.zeros((pad,), jnp.int32)
        src = jnp.concatenate([src, zi])
        dst = jnp.concatenate([dst, zi])
        w = jnp.concatenate([edge_weight, jnp.zeros((pad,), jnp.float32)])
    else:
        w = edge_weight
    src2 = jnp.stack([src, src + n]).reshape(_NC, _NS, nchunk, _K)
    dst_r = dst.reshape(_NS, nchunk, _K)
    w_r = w.reshape(_NS, nchunk, _K)

    spmm = _make_spmm(nchunk)

    p, h1 = _tc0(features, W_lin, b_lin, W1)
    s1 = spmm(h1.reshape(_NC * n, _D), src2, dst_r, w_r)
    g1, h2 = _tc_mid(s1, b1, W2)
    s2 = spmm(h2.reshape(_NC * n, _D), src2, dst_r, w_r)
    g2, h3 = _tc_mid(s2, b2, W2)
    s3 = spmm(h3.reshape(_NC * n, _D), src2, dst_r, w_r)
    g3 = _tc_bias(s3, b2)
    return jnp.concatenate([p, g1, g2, g3], axis=1)


# speedup vs baseline: 2.5988x; 2.5988x over previous
import functools

import jax
import jax.numpy as jnp
from jax import lax
from jax.experimental import pallas as pl
from jax.experimental.pallas import tpu as pltpu
from jax.experimental.pallas import tpu_sc as plsc

_N = 10000
_D = 128
_NS = 16
_NC = 2
_K = 128
_ROW_BLK = 80
_NBLK = _N // _ROW_BLK


def _cdiv(a, b):
    return (a + b - 1) // b


_RB = 1000
_GRID = _N // _RB


def _tc0_body(f_ref, wl_ref, bl_ref, w1_ref, p_ref, h_ref):
    p = jnp.dot(f_ref[...], wl_ref[...], preferred_element_type=jnp.float32)
    p = p + bl_ref[...]
    p_ref[...] = p
    x = jax.nn.sigmoid(p)
    h = jnp.dot(x, w1_ref[...], preferred_element_type=jnp.float32)
    h_ref[0] = h[:, :_D]
    h_ref[1] = h[:, _D:]


def _tc0(features, W_lin, b_lin, W1):
    return pl.pallas_call(
        _tc0_body,
        grid=(_GRID,),
        in_specs=[
            pl.BlockSpec((_RB, 128), lambda i: (i, 0)),
            pl.BlockSpec((128, 256), lambda i: (0, 0)),
            pl.BlockSpec((1, 256), lambda i: (0, 0)),
            pl.BlockSpec((256, 256), lambda i: (0, 0)),
        ],
        out_specs=[
            pl.BlockSpec((_RB, 256), lambda i: (i, 0)),
            pl.BlockSpec((2, _RB, _D), lambda i: (0, i, 0)),
        ],
        out_shape=[
            jax.ShapeDtypeStruct((_N, 256), jnp.float32),
            jax.ShapeDtypeStruct((2, _N, _D), jnp.float32),
        ],
    )(features, W_lin, b_lin.reshape(1, 256), W1)


def _tc_mid_body(s_ref, b_ref, w_ref, g_ref, h_ref):
    g = jnp.concatenate([s_ref[0], s_ref[1]], axis=1) + b_ref[...]
    g_ref[...] = g
    x = jnp.maximum(g, 0.0)
    h = jnp.dot(x, w_ref[...], preferred_element_type=jnp.float32)
    h_ref[0] = h[:, :_D]
    h_ref[1] = h[:, _D:]


def _tc_mid(s_split, b, W):
    return pl.pallas_call(
        _tc_mid_body,
        grid=(_GRID,),
        in_specs=[
            pl.BlockSpec((2, _RB, _D), lambda i: (0, i, 0)),
            pl.BlockSpec((1, 256), lambda i: (0, 0)),
            pl.BlockSpec((256, 256), lambda i: (0, 0)),
        ],
        out_specs=[
            pl.BlockSpec((_RB, 256), lambda i: (i, 0)),
            pl.BlockSpec((2, _RB, _D), lambda i: (0, i, 0)),
        ],
        out_shape=[
            jax.ShapeDtypeStruct((_N, 256), jnp.float32),
            jax.ShapeDtypeStruct((2, _N, _D), jnp.float32),
        ],
    )(s_split, b.reshape(1, 256), W)


def _tc_bias_body(s_ref, b_ref, g_ref):
    g_ref[...] = jnp.concatenate([s_ref[0], s_ref[1]], axis=1) + b_ref[...]


def _tc_bias(s_split, b):
    return pl.pallas_call(
        _tc_bias_body,
        grid=(_GRID,),
        in_specs=[
            pl.BlockSpec((2, _RB, _D), lambda i: (0, i, 0)),
            pl.BlockSpec((1, 256), lambda i: (0, 0)),
        ],
        out_specs=pl.BlockSpec((_RB, 256), lambda i: (i, 0)),
        out_shape=jax.ShapeDtypeStruct((_N, 256), jnp.float32),
    )(s_split, b.reshape(1, 256))


def _spmm_body(h_hbm, src_hbm, dst_hbm, w_hbm, out_hbm,
               acc, src_t, dst_t, w_t, rows_v, sem, nchunk):
    c = lax.axis_index("c")
    s = lax.axis_index("s")

    zero = jnp.zeros((16,), jnp.float32)

    def zrow(r, carry):
        for v in range(8):
            rows_v[r, pl.ds(v * 16, 16)] = zero
        return carry

    lax.fori_loop(0, _K, zrow, 0)
    for b in range(_cdiv(_NBLK, _NS)):
        blk = s + _NS * b

        @pl.when(blk < _NBLK)
        def _():
            pltpu.sync_copy(
                rows_v.at[pl.ds(0, _ROW_BLK)],
                acc.at[pl.ds(blk * _ROW_BLK, _ROW_BLK)],
            )
    plsc.subcore_barrier()

    def grp(jo, carry):
        pltpu.sync_copy(src_hbm.at[c, s, pl.ds(jo * 8, 8)], src_t)
        pltpu.sync_copy(dst_hbm.at[s, pl.ds(jo * 8, 8)], dst_t)
        pltpu.sync_copy(w_hbm.at[s, pl.ds(jo * 8, 8)], w_t)

        def one(ji, c1):
            pltpu.async_copy(h_hbm.at[src_t.at[ji]], rows_v, sem).wait()

            def group(g, c2):
                wvec = w_t[ji, pl.ds(g * 16, 16)]
                for el in range(16):
                    ws = wvec[el]
                    e = g * 16 + el
                    for v in range(8):
                        sl = pl.ds(v * 16, 16)
                        rows_v[e, sl] = rows_v[e, sl] * ws
                return c2

            lax.fori_loop(0, _K // 16, group, 0)
            pltpu.sync_copy(rows_v, acc.at[dst_t.at[ji]], add=True)
            return c1

        lax.fori_loop(0, 8, one, 0)
        return carry

    lax.fori_loop(0, nchunk // 8, grp, 0)
    plsc.subcore_barrier()

    for b in range(_cdiv(_NBLK, _NS)):
        blk = s + _NS * b

        @pl.when(blk < _NBLK)
        def _():
            pltpu.sync_copy(
                acc.at[pl.ds(blk * _ROW_BLK, _ROW_BLK)],
                out_hbm.at[c, pl.ds(blk * _ROW_BLK, _ROW_BLK)],
            )


def _make_spmm(nchunk):
    return functools.partial(
        pl.kernel,
        out_type=jax.ShapeDtypeStruct((_NC, _N, _D), jnp.float32),
        mesh=plsc.VectorSubcoreMesh(core_axis_name="c", subcore_axis_name="s"),
        scratch_types=[
            pltpu.VMEM_SHARED((_N, _D), jnp.float32),
            pltpu.VMEM((8, _K), jnp.int32),
            pltpu.VMEM((8, _K), jnp.int32),
            pltpu.VMEM((8, _K), jnp.float32),
            pltpu.VMEM((_K, _D), jnp.float32),
            pltpu.SemaphoreType.DMA,
        ],
    )(functools.partial(_spmm_body, nchunk=nchunk))


def kernel(features, edge_index, edge_weight, W_lin, b_lin, W1, b1, W2, b2):
    n = features.shape[0]
    e = edge_index.shape[1]
    nchunk = 8 * _cdiv(e, _NS * _K * 8)
    e_pad = _NS * nchunk * _K

    src = edge_index[0]
    dst = edge_index[1]
    pad = e_pad - e
    if pad:
        zi = jnp.zeros((pad,), jnp.int32)
        src = jnp.concatenate([src, zi])
        dst = jnp.concatenate([dst, zi])
        w = jnp.concatenate([edge_weight, jnp.zeros((pad,), jnp.float32)])
    else:
        w = edge_weight
    src2 = jnp.stack([src, src + n]).reshape(_NC, _NS, nchunk, _K)
    dst_r = dst.reshape(_NS, nchunk, _K)
    w_r = w.reshape(_NS, nchunk, _K)

    spmm = _make_spmm(nchunk)

    p, h1 = _tc0(features, W_lin, b_lin, W1)
    s1 = spmm(h1.reshape(_NC * n, _D), src2, dst_r, w_r)
    g1, h2 = _tc_mid(s1, b1, W2)
    s2 = spmm(h2.reshape(_NC * n, _D), src2, dst_r, w_r)
    g2, h3 = _tc_mid(s2, b2, W2)
    s3 = spmm(h3.reshape(_NC * n, _D), src2, dst_r, w_r)
    g3 = _tc_bias(s3, b2)
    return jnp.concatenate([p, g1, g2, g3], axis=1)
